# baseline (speedup 1.0000x reference)
import functools

import jax
import jax.numpy as jnp
from jax import lax
from jax.experimental import pallas as pl
from jax.experimental.pallas import tpu as pltpu
from jax.experimental.pallas import tpu_sc as plsc

NC = 2
NS = 16
NW = NC * NS
LANES = 16
GATHER = 128
GPC = 4
CHUNK = GATHER * GPC
SCALE = 8.0


@functools.partial(jax.jit, static_argnums=(2, 3))
def _embed(idx, table, n_chunks, d):
  b = idx.shape[0] * idx.shape[1] * idx.shape[2]
  per_w = b // NW
  mesh = plsc.VectorSubcoreMesh(core_axis_name="c", subcore_axis_name="s")

  @functools.partial(
      pl.kernel,
      mesh=mesh,
      out_type=jax.ShapeDtypeStruct((b, d), jnp.float32),
      scratch_types=[
          pltpu.VMEM((n_chunks * GPC, GATHER), jnp.int32),
          pltpu.VMEM((CHUNK, d), jnp.float32),
          pltpu.SemaphoreType.DMA,
      ],
      compiler_params=pltpu.CompilerParams(use_tc_tiling_on_sc=False),
  )
  def k(idx_hbm, table_hbm, out_hbm, idx_v, rows_v, sem):
    wid = lax.axis_index("s") * NC + lax.axis_index("c")
    pltpu.sync_copy(idx_hbm.at[wid], idx_v)

    def chunk_body(c, carry):
      copies = [
          pltpu.async_copy(
              table_hbm.at[idx_v.at[c * GPC + g]],
              rows_v.at[pl.ds(g * GATHER, GATHER)],
              sem,
          )
          for g in range(GPC)
      ]
      for cp in copies:
        cp.wait()

      def scale_body(i, carry2):
        for r in range(4):
          for j in range(d // LANES):
            sl = (i * 4 + r, pl.ds(j * LANES, LANES))
            rows_v[sl] = rows_v[sl] * SCALE
        return carry2

      lax.fori_loop(0, CHUNK // 4, scale_body, 0)

      pltpu.sync_copy(
          rows_v, out_hbm.at[pl.ds(wid * per_w + c * CHUNK, CHUNK)]
      )
      return carry

    lax.fori_loop(0, n_chunks, chunk_body, 0)

  return k(idx, table)


def kernel(input_tensor, table):
  batch, hist = input_tensor.shape
  vocab, d = table.shape
  b = batch * hist
  per_w = b // NW
  n_chunks = per_w // CHUNK
  idx = input_tensor.reshape(-1).astype(jnp.int32).reshape(NW, per_w // GATHER, GATHER)
  out = _embed(idx, table, n_chunks, d)
  return out.reshape(batch, hist, d)
